# baseline (speedup 1.0000x reference)
import functools

import jax
import jax.numpy as jnp
from jax.experimental import pallas as pl
from jax.experimental.pallas import tpu as pltpu

_P = jax.lax.Precision.DEFAULT


def _dot(a, b):
    return jax.lax.dot_general(
        a, b, (((1,), (0,)), ((), ())),
        preferred_element_type=jnp.float32, precision=_P)


def _fused(nk, valid_last, gene_ref, adj_ref, x_ref, ws_ref, bs_ref, wf_ref,
           bf_ref, w1_ref, b1_ref, w2_ref, b2_ref, w3_ref, b3_ref,
           out_ref, acc_ref, hf_ref):
    k = pl.program_id(0)

    ax = _dot(gene_ref[...], x_ref[...])
    hf_ref[...] = _dot(jnp.tanh(_dot(ax, ws_ref[...]) + bs_ref[...]),
                       wf_ref[...])

    if valid_last < hf_ref.shape[0]:
        @pl.when(k == nk - 1)
        def _mask_ragged():
            hf_ref[valid_last:, :] = jnp.zeros_like(hf_ref[valid_last:, :])
            adj_ref[:, valid_last:] = jnp.zeros_like(adj_ref[:, valid_last:])

    contrib = _dot(adj_ref[...], hf_ref[...])

    @pl.when(k == 0)
    def _init():
        acc_ref[...] = contrib

    @pl.when(k > 0)
    def _accum():
        acc_ref[...] = acc_ref[...] + contrib

    @pl.when(k == nk - 1)
    def _head():
        h = jnp.tanh(acc_ref[...] + bf_ref[...])
        h = jnp.tanh(_dot(h, w1_ref[...]) + b1_ref[...])
        h = jnp.tanh(_dot(h, w2_ref[...]) + b2_ref[...])
        out_ref[...] = _dot(h, w3_ref[...]) + b3_ref[...]


def kernel(x, adj, gene_adj, W_s, b_s, W_f, b_f, W1, b1, W2, b2, W3, b3):
    n, f = x.shape
    f1 = W1.shape[1]
    f2 = W2.shape[1]
    nc = W3.shape[1]
    bk = 128
    nk = -(-n // bk)
    valid_last = n - (nk - 1) * bk

    def _const(shape):
        return pl.BlockSpec(shape, lambda k: (0, 0))

    body = functools.partial(_fused, nk, valid_last)

    out = pl.pallas_call(
        body,
        grid=(nk,),
        in_specs=[
            pl.BlockSpec((bk, n), lambda k: (k, 0)),
            pl.BlockSpec((n, bk), lambda k: (0, k)),
            _const((n, f)),
            _const((f, f)),
            _const((1, f)),
            _const((f, f)),
            _const((1, f)),
            _const((f, f1)),
            _const((1, f1)),
            _const((f1, f2)),
            _const((1, f2)),
            _const((f2, nc)),
            _const((1, nc)),
        ],
        out_specs=pl.BlockSpec((n, nc), lambda k: (0, 0)),
        out_shape=jax.ShapeDtypeStruct((n, nc), jnp.float32),
        scratch_shapes=[
            pltpu.VMEM((n, f), jnp.float32),
            pltpu.VMEM((bk, f), jnp.float32),
        ],
        compiler_params=pltpu.CompilerParams(
            dimension_semantics=("arbitrary",),
            vmem_limit_bytes=62 * 1024 * 1024,
        ),
    )(gene_adj, adj, x, W_s, b_s.reshape(1, f), W_f, b_f.reshape(1, f),
      W1, b1.reshape(1, f1), W2, b2.reshape(1, f2), W3, b3.reshape(1, nc))
    return out
